# baseline (speedup 1.0000x reference)
import functools

import jax
import jax.numpy as jnp
from jax import lax
from jax.experimental import pallas as pl
from jax.experimental.pallas import tpu as pltpu
from jax.experimental.pallas import tpu_sc as plsc

N = 10000
D = 128
E = 320000

NC = 2
NS = 16
NW = NC * NS

K = 128
NBLK = 80
NBLK_H = NBLK // 2
EPW = NBLK * K
EPAD = NW * EPW
NPAD = 10240
SLAB = NPAD // NS

_mesh = plsc.VectorSubcoreMesh(core_axis_name="c", subcore_axis_name="s")


@functools.partial(
    pl.kernel,
    out_type=jax.ShapeDtypeStruct((NC, NPAD), jnp.float32),
    mesh=_mesh,
    scratch_types=[
        pltpu.VMEM((NBLK, K), jnp.int32),
        pltpu.VMEM((K,), jnp.float32),
        pltpu.VMEM((SLAB,), jnp.float32),
        pltpu.VMEM_SHARED((NPAD,), jnp.float32),
    ],
)
def _deg_kernel(dst3, out, dstv, onesv, slabv, dshared):
    c = lax.axis_index("c")
    s = lax.axis_index("s")
    wid = s * NC + c

    def initz(i, _):
        slabv[pl.ds(i * 16, 16)] = jnp.zeros((16,), jnp.float32)
        return 0

    lax.fori_loop(0, SLAB // 16, initz, 0)

    def inito(i, _):
        onesv[pl.ds(i * 16, 16)] = jnp.ones((16,), jnp.float32)
        return 0

    lax.fori_loop(0, K // 16, inito, 0)

    pltpu.sync_copy(slabv, dshared.at[pl.ds(s * SLAB, SLAB)])
    pltpu.sync_copy(dst3.at[wid], dstv)
    plsc.subcore_barrier()

    def step(b, _):
        pltpu.sync_copy(onesv, dshared.at[dstv.at[b]], add=True)
        return 0

    lax.fori_loop(0, NBLK, step, 0)
    plsc.subcore_barrier()

    pltpu.sync_copy(dshared.at[pl.ds(s * SLAB, SLAB)], slabv)
    pltpu.sync_copy(slabv, out.at[c, pl.ds(s * SLAB, SLAB)])


@functools.partial(
    pl.kernel,
    out_type=jax.ShapeDtypeStruct((NC, NPAD, D), jnp.float32),
    mesh=_mesh,
    scratch_types=[
        pltpu.VMEM((NBLK_H, K), jnp.int32),
        pltpu.VMEM((NBLK_H, K), jnp.int32),
        pltpu.VMEM((K, D), jnp.float32),
        pltpu.VMEM((K, D), jnp.float32),
        pltpu.VMEM_SHARED((NPAD, D), jnp.float32),
        pltpu.SemaphoreType.DMA,
        pltpu.SemaphoreType.DMA,
        pltpu.SemaphoreType.DMA,
        pltpu.SemaphoreType.DMA,
    ],
)
def _agg_kernel(g_hbm, src3, dst3, zeros_hbm, out, srcv, dstv, rows0, rows1,
                acc, sem0, sem1, sems0, sems1):
    c = lax.axis_index("c")
    s = lax.axis_index("s")
    wid = s * NC + c

    pltpu.sync_copy(zeros_hbm, rows0)
    for j in range(SLAB // K):
        pltpu.sync_copy(rows0, acc.at[pl.ds(s * SLAB + j * K, K)])
    plsc.subcore_barrier()

    for h in range(NBLK // NBLK_H):
        pltpu.sync_copy(src3.at[wid, pl.ds(h * NBLK_H, NBLK_H)], srcv)
        pltpu.sync_copy(dst3.at[wid, pl.ds(h * NBLK_H, NBLK_H)], dstv)
        pltpu.async_copy(g_hbm.at[srcv.at[0]], rows0, sem0)
        pltpu.async_copy(g_hbm.at[srcv.at[1]], rows1, sem1)

        def step2(i, _):
            b = i * 2
            pltpu.make_async_copy(g_hbm.at[srcv.at[b]], rows0, sem0).wait()
            pltpu.async_copy(rows0, acc.at[dstv.at[b]], sems0, add=True)
            pltpu.make_async_copy(g_hbm.at[srcv.at[b + 1]], rows1, sem1).wait()
            pltpu.async_copy(rows1, acc.at[dstv.at[b + 1]], sems1, add=True)

            pltpu.make_async_copy(rows0, acc.at[dstv.at[b]], sems0).wait()

            @pl.when(b + 2 < NBLK_H)
            def _():
                pltpu.async_copy(g_hbm.at[srcv.at[b + 2]], rows0, sem0)

            pltpu.make_async_copy(rows1, acc.at[dstv.at[b + 1]], sems1).wait()

            @pl.when(b + 3 < NBLK_H)
            def _():
                pltpu.async_copy(g_hbm.at[srcv.at[b + 3]], rows1, sem1)

            return 0

        lax.fori_loop(0, NBLK_H // 2, step2, 0)
    plsc.subcore_barrier()

    for j in range(SLAB // K):
        r0 = s * SLAB + j * K
        pltpu.sync_copy(acc.at[pl.ds(r0, K)], rows0)
        pltpu.sync_copy(rows0, out.at[c, pl.ds(r0, K)])


_R = 2048
_G = NPAD // _R


def _b0_body(deg_ref, x_ref, w_ref, g_ref, dinv_ref):
    d = deg_ref[:, 0:1] + deg_ref[:, 1:2] + 1.0
    dinv = lax.rsqrt(d)
    m = jnp.dot(x_ref[...], w_ref[...], preferred_element_type=jnp.float32)
    g_ref[...] = dinv * m
    dinv_ref[...] = dinv


_b0_call = pl.pallas_call(
    _b0_body,
    grid=(_G,),
    in_specs=[
        pl.BlockSpec((_R, 2), lambda i: (i, 0)),
        pl.BlockSpec((_R, D), lambda i: (i, 0)),
        pl.BlockSpec((D, D), lambda i: (0, 0)),
    ],
    out_specs=[
        pl.BlockSpec((_R, D), lambda i: (i, 0)),
        pl.BlockSpec((_R, 1), lambda i: (i, 0)),
    ],
    out_shape=[
        jax.ShapeDtypeStruct((NPAD, D), jnp.float32),
        jax.ShapeDtypeStruct((NPAD, 1), jnp.float32),
    ],
)


def _mid_body(acc_ref, g_ref, dinv_ref, b_ref, w_ref, o_ref):
    accsum = acc_ref[0] + acc_ref[1]
    dinv = dinv_ref[...]
    h = jnp.maximum(dinv * (accsum + g_ref[...]) + b_ref[...], 0.0)
    o_ref[...] = dinv * jnp.dot(h, w_ref[...], preferred_element_type=jnp.float32)


_mid_call = pl.pallas_call(
    _mid_body,
    grid=(_G,),
    in_specs=[
        pl.BlockSpec((2, _R, D), lambda i: (0, i, 0)),
        pl.BlockSpec((_R, D), lambda i: (i, 0)),
        pl.BlockSpec((_R, 1), lambda i: (i, 0)),
        pl.BlockSpec((1, D), lambda i: (0, 0)),
        pl.BlockSpec((D, D), lambda i: (0, 0)),
    ],
    out_specs=pl.BlockSpec((_R, D), lambda i: (i, 0)),
    out_shape=jax.ShapeDtypeStruct((NPAD, D), jnp.float32),
)


def _final_body(acc_ref, g_ref, dinv_ref, b_ref, o_ref):
    accsum = acc_ref[0] + acc_ref[1]
    o_ref[...] = dinv_ref[...] * (accsum + g_ref[...]) + b_ref[...]


_final_call = pl.pallas_call(
    _final_body,
    grid=(_G,),
    in_specs=[
        pl.BlockSpec((2, _R, D), lambda i: (0, i, 0)),
        pl.BlockSpec((_R, D), lambda i: (i, 0)),
        pl.BlockSpec((_R, 1), lambda i: (i, 0)),
        pl.BlockSpec((1, D), lambda i: (0, 0)),
    ],
    out_specs=pl.BlockSpec((_R, D), lambda i: (i, 0)),
    out_shape=jax.ShapeDtypeStruct((NPAD, D), jnp.float32),
)


def kernel(x, edge_index, W0, b0, W1, b1, W2, b2):
    src = edge_index[0].astype(jnp.int32)
    dst = edge_index[1].astype(jnp.int32)
    epad = EPAD - E
    src3 = jnp.concatenate([src, jnp.zeros((epad,), jnp.int32)]).reshape(
        NW, NBLK, K)
    dst3 = jnp.concatenate([dst, jnp.full((epad,), N, jnp.int32)]).reshape(
        NW, NBLK, K)
    xp = jnp.pad(x, ((0, NPAD - N), (0, 0)))
    zeros_blk = jnp.zeros((K, D), jnp.float32)
    b0_2 = b0[None, :]
    b1_2 = b1[None, :]
    b2_2 = b2[None, :]

    degp = _deg_kernel(dst3)
    deg_t = degp.T
    g0, dinv = _b0_call(deg_t, xp, W0)
    acc0 = _agg_kernel(g0, src3, dst3, zeros_blk)
    g1 = _mid_call(acc0, g0, dinv, b0_2, W1)
    acc1 = _agg_kernel(g1, src3, dst3, zeros_blk)
    g2 = _mid_call(acc1, g1, dinv, b1_2, W2)
    acc2 = _agg_kernel(g2, src3, dst3, zeros_blk)
    out = _final_call(acc2, g2, dinv, b2_2)
    return out[:N]


# speedup vs baseline: 1.0874x; 1.0874x over previous
import functools

import jax
import jax.numpy as jnp
from jax import lax
from jax.experimental import pallas as pl
from jax.experimental.pallas import tpu as pltpu
from jax.experimental.pallas import tpu_sc as plsc

N = 10000
D = 128
E = 320000

NC = 2
NS = 16
NW = NC * NS

K = 128
B0 = 128
B1 = 32
CHUNK = 40
CH0 = (40, 40, 40, 8)
CH1 = (32,)
NBLOCKS = NS * (B0 + B1)
NBLOCKS_AL = NBLOCKS + 64
EPAD = NBLOCKS * K
NPAD = 10240
SLAB = NPAD // NS

_mesh = plsc.VectorSubcoreMesh(core_axis_name="c", subcore_axis_name="s")


@functools.partial(
    pl.kernel,
    out_type=jax.ShapeDtypeStruct((NC, NPAD), jnp.float32),
    mesh=_mesh,
    scratch_types=[
        pltpu.VMEM((CHUNK, K), jnp.int32),
        pltpu.VMEM((K,), jnp.float32),
        pltpu.VMEM((SLAB,), jnp.float32),
        pltpu.VMEM_SHARED((NPAD,), jnp.float32),
    ],
)
def _deg_kernel(dst2, out, dstv, onesv, slabv, dshared):
    c = lax.axis_index("c")
    s = lax.axis_index("s")

    def initz(i, _):
        slabv[pl.ds(i * 16, 16)] = jnp.zeros((16,), jnp.float32)
        return 0

    lax.fori_loop(0, SLAB // 16, initz, 0)

    def inito(i, _):
        onesv[pl.ds(i * 16, 16)] = jnp.ones((16,), jnp.float32)
        return 0

    lax.fori_loop(0, K // 16, inito, 0)

    pltpu.sync_copy(slabv, dshared.at[pl.ds(s * SLAB, SLAB)])
    plsc.subcore_barrier()

    def _scan(base_blk, chunks):
        off = 0
        for m in chunks:
            pltpu.sync_copy(dst2.at[pl.ds(base_blk + off, m)],
                            dstv.at[pl.ds(0, m)])

            def step(b, _):
                pltpu.sync_copy(onesv, dshared.at[dstv.at[b]], add=True)
                return 0

            lax.fori_loop(0, m, step, 0)
            off += m

    @pl.when(c == 0)
    def _():
        _scan(s * B0, CH0)

    @pl.when(c == 1)
    def _():
        _scan(NS * B0 + s * B1, CH1)

    plsc.subcore_barrier()
    pltpu.sync_copy(dshared.at[pl.ds(s * SLAB, SLAB)], slabv)
    pltpu.sync_copy(slabv, out.at[c, pl.ds(s * SLAB, SLAB)])


@functools.partial(
    pl.kernel,
    out_type=jax.ShapeDtypeStruct((NC, NPAD, D), jnp.float32),
    mesh=_mesh,
    scratch_types=[
        pltpu.VMEM((CHUNK, K), jnp.int32),
        pltpu.VMEM((CHUNK, K), jnp.int32),
        pltpu.VMEM((K, D), jnp.float32),
        pltpu.VMEM((K, D), jnp.float32),
        pltpu.VMEM_SHARED((NPAD, D), jnp.float32),
        pltpu.SemaphoreType.DMA,
        pltpu.SemaphoreType.DMA,
        pltpu.SemaphoreType.DMA,
        pltpu.SemaphoreType.DMA,
    ],
)
def _agg_kernel(g_hbm, src2, dst2, zeros_hbm, out, srcv, dstv, rows0, rows1,
                acc, sem0, sem1, sems0, sems1):
    c = lax.axis_index("c")
    s = lax.axis_index("s")

    pltpu.sync_copy(zeros_hbm, rows0)
    for j in range(SLAB // K):
        pltpu.sync_copy(rows0, acc.at[pl.ds(s * SLAB + j * K, K)])
    plsc.subcore_barrier()

    def _chunk(blk0, m):
        pltpu.sync_copy(src2.at[pl.ds(blk0, m)], srcv.at[pl.ds(0, m)])
        pltpu.sync_copy(dst2.at[pl.ds(blk0, m)], dstv.at[pl.ds(0, m)])
        pltpu.async_copy(g_hbm.at[srcv.at[0]], rows0, sem0)
        pltpu.async_copy(g_hbm.at[srcv.at[1]], rows1, sem1)

        def step2(i, _):
            b = i * 2
            pltpu.make_async_copy(g_hbm.at[srcv.at[b]], rows0, sem0).wait()
            pltpu.async_copy(rows0, acc.at[dstv.at[b]], sems0, add=True)
            pltpu.make_async_copy(g_hbm.at[srcv.at[b + 1]], rows1, sem1).wait()
            pltpu.async_copy(rows1, acc.at[dstv.at[b + 1]], sems1, add=True)

            pltpu.make_async_copy(rows0, acc.at[dstv.at[b]], sems0).wait()

            @pl.when(b + 2 < m)
            def _():
                pltpu.async_copy(g_hbm.at[srcv.at[b + 2]], rows0, sem0)

            pltpu.make_async_copy(rows1, acc.at[dstv.at[b + 1]], sems1).wait()

            @pl.when(b + 3 < m)
            def _():
                pltpu.async_copy(g_hbm.at[srcv.at[b + 3]], rows1, sem1)

            return 0

        lax.fori_loop(0, m // 2, step2, 0)

    def _run(base_blk, chunks):
        off = 0
        for m in chunks:
            _chunk(base_blk + off, m)
            off += m

    @pl.when(c == 0)
    def _():
        _run(s * B0, CH0)

    @pl.when(c == 1)
    def _():
        _run(NS * B0 + s * B1, CH1)

    plsc.subcore_barrier()

    for j in range(SLAB // K):
        r0 = s * SLAB + j * K
        pltpu.sync_copy(acc.at[pl.ds(r0, K)], rows0)
        pltpu.sync_copy(rows0, out.at[c, pl.ds(r0, K)])


_R = 2048
_G = NPAD // _R


def _b0_body(deg_ref, x_ref, w_ref, g_ref, dinv_ref):
    d = deg_ref[:, 0:1] + deg_ref[:, 1:2] + 1.0
    dinv = lax.rsqrt(d)
    m = jnp.dot(x_ref[...], w_ref[...], preferred_element_type=jnp.float32)
    g_ref[...] = dinv * m
    dinv_ref[...] = dinv


_b0_call = pl.pallas_call(
    _b0_body,
    grid=(_G,),
    in_specs=[
        pl.BlockSpec((_R, 2), lambda i: (i, 0)),
        pl.BlockSpec((_R, D), lambda i: (i, 0)),
        pl.BlockSpec((D, D), lambda i: (0, 0)),
    ],
    out_specs=[
        pl.BlockSpec((_R, D), lambda i: (i, 0)),
        pl.BlockSpec((_R, 1), lambda i: (i, 0)),
    ],
    out_shape=[
        jax.ShapeDtypeStruct((NPAD, D), jnp.float32),
        jax.ShapeDtypeStruct((NPAD, 1), jnp.float32),
    ],
)


def _mid_body(acc_ref, g_ref, dinv_ref, b_ref, w_ref, o_ref):
    accsum = acc_ref[0] + acc_ref[1]
    dinv = dinv_ref[...]
    h = jnp.maximum(dinv * (accsum + g_ref[...]) + b_ref[...], 0.0)
    o_ref[...] = dinv * jnp.dot(h, w_ref[...], preferred_element_type=jnp.float32)


_mid_call = pl.pallas_call(
    _mid_body,
    grid=(_G,),
    in_specs=[
        pl.BlockSpec((2, _R, D), lambda i: (0, i, 0)),
        pl.BlockSpec((_R, D), lambda i: (i, 0)),
        pl.BlockSpec((_R, 1), lambda i: (i, 0)),
        pl.BlockSpec((1, D), lambda i: (0, 0)),
        pl.BlockSpec((D, D), lambda i: (0, 0)),
    ],
    out_specs=pl.BlockSpec((_R, D), lambda i: (i, 0)),
    out_shape=jax.ShapeDtypeStruct((NPAD, D), jnp.float32),
)


def _final_body(acc_ref, g_ref, dinv_ref, b_ref, o_ref):
    accsum = acc_ref[0] + acc_ref[1]
    o_ref[...] = dinv_ref[...] * (accsum + g_ref[...]) + b_ref[...]


_final_call = pl.pallas_call(
    _final_body,
    grid=(_G,),
    in_specs=[
        pl.BlockSpec((2, _R, D), lambda i: (0, i, 0)),
        pl.BlockSpec((_R, D), lambda i: (i, 0)),
        pl.BlockSpec((_R, 1), lambda i: (i, 0)),
        pl.BlockSpec((1, D), lambda i: (0, 0)),
    ],
    out_specs=pl.BlockSpec((_R, D), lambda i: (i, 0)),
    out_shape=jax.ShapeDtypeStruct((NPAD, D), jnp.float32),
)


def kernel(x, edge_index, W0, b0, W1, b1, W2, b2):
    src = edge_index[0].astype(jnp.int32)
    dst = edge_index[1].astype(jnp.int32)
    epad = NBLOCKS_AL * K - E
    src3 = jnp.concatenate([src, jnp.zeros((epad,), jnp.int32)]).reshape(
        NBLOCKS_AL, K)
    dst3 = jnp.concatenate([dst, jnp.full((epad,), N, jnp.int32)]).reshape(
        NBLOCKS_AL, K)
    xp = jnp.pad(x, ((0, NPAD - N), (0, 0)))
    zeros_blk = jnp.zeros((K, D), jnp.float32)
    b0_2 = b0[None, :]
    b1_2 = b1[None, :]
    b2_2 = b2[None, :]

    degp = _deg_kernel(dst3)
    deg_t = degp.T
    g0, dinv = _b0_call(deg_t, xp, W0)
    acc0 = _agg_kernel(g0, src3, dst3, zeros_blk)
    g1 = _mid_call(acc0, g0, dinv, b0_2, W1)
    acc1 = _agg_kernel(g1, src3, dst3, zeros_blk)
    g2 = _mid_call(acc1, g1, dinv, b1_2, W2)
    acc2 = _agg_kernel(g2, src3, dst3, zeros_blk)
    out = _final_call(acc2, g2, dinv, b2_2)
    return out[:N]


# speedup vs baseline: 1.1935x; 1.0976x over previous
import functools

import jax
import jax.numpy as jnp
from jax import lax
from jax.experimental import pallas as pl
from jax.experimental.pallas import tpu as pltpu
from jax.experimental.pallas import tpu_sc as plsc

N = 10000
D = 128
E = 320000

NC = 2
NS = 16
NW = NC * NS

K = 128
B0 = 128
B1 = 32
CHUNK = 40
CH0 = (40, 40, 40, 8)
CH1 = (32,)
NBLOCKS = NS * (B0 + B1)
NBLOCKS_AL = NBLOCKS + 64
EPAD = NBLOCKS * K
NPAD = 10240
SLAB = NPAD // NS

_mesh = plsc.VectorSubcoreMesh(core_axis_name="c", subcore_axis_name="s")


@functools.partial(
    pl.kernel,
    out_type=jax.ShapeDtypeStruct((NC, NPAD), jnp.float32),
    mesh=_mesh,
    scratch_types=[
        pltpu.VMEM((CHUNK, K), jnp.int32),
        pltpu.VMEM((K,), jnp.float32),
        pltpu.VMEM((SLAB,), jnp.float32),
        pltpu.VMEM_SHARED((NPAD,), jnp.float32),
    ],
)
def _deg_kernel(dst2, out, dstv, onesv, slabv, dshared):
    c = lax.axis_index("c")
    s = lax.axis_index("s")

    def initz(i, _):
        slabv[pl.ds(i * 16, 16)] = jnp.zeros((16,), jnp.float32)
        return 0

    lax.fori_loop(0, SLAB // 16, initz, 0)

    def inito(i, _):
        onesv[pl.ds(i * 16, 16)] = jnp.ones((16,), jnp.float32)
        return 0

    lax.fori_loop(0, K // 16, inito, 0)

    pltpu.sync_copy(slabv, dshared.at[pl.ds(s * SLAB, SLAB)])
    plsc.subcore_barrier()

    def _scan(base_blk, chunks):
        off = 0
        for m in chunks:
            pltpu.sync_copy(dst2.at[pl.ds(base_blk + off, m)],
                            dstv.at[pl.ds(0, m)])

            def step(b, _):
                pltpu.sync_copy(onesv, dshared.at[dstv.at[b]], add=True)
                return 0

            lax.fori_loop(0, m, step, 0)
            off += m

    @pl.when(c == 0)
    def _():
        _scan(s * B0, CH0)

    @pl.when(c == 1)
    def _():
        _scan(NS * B0 + s * B1, CH1)

    plsc.subcore_barrier()
    pltpu.sync_copy(dshared.at[pl.ds(s * SLAB, SLAB)], slabv)
    pltpu.sync_copy(slabv, out.at[c, pl.ds(s * SLAB, SLAB)])


@functools.partial(
    pl.kernel,
    out_type=jax.ShapeDtypeStruct((NC, NPAD, D), jnp.float32),
    mesh=_mesh,
    scratch_types=[
        pltpu.VMEM((CHUNK, K), jnp.int32),
        pltpu.VMEM((CHUNK, K), jnp.int32),
        pltpu.VMEM((K, D), jnp.float32),
        pltpu.VMEM((K, D), jnp.float32),
        pltpu.VMEM_SHARED((NPAD, D), jnp.float32),
        pltpu.SemaphoreType.DMA,
        pltpu.SemaphoreType.DMA,
        pltpu.SemaphoreType.DMA,
        pltpu.SemaphoreType.DMA,
    ],
)
def _agg_kernel(g_hbm, src2, dst2, zeros_hbm, out, srcv, dstv, rows0, rows1,
                acc, sem0, sem1, sems0, sems1):
    c = lax.axis_index("c")
    s = lax.axis_index("s")

    pltpu.sync_copy(zeros_hbm.at[c * NS + s], rows0)
    for j in range(SLAB // K):
        pltpu.sync_copy(rows0, acc.at[pl.ds(s * SLAB + j * K, K)])
    plsc.subcore_barrier()

    def _chunk(blk0, m):
        pltpu.sync_copy(src2.at[pl.ds(blk0, m)], srcv.at[pl.ds(0, m)])
        pltpu.sync_copy(dst2.at[pl.ds(blk0, m)], dstv.at[pl.ds(0, m)])
        pltpu.async_copy(g_hbm.at[srcv.at[0]], rows0, sem0)
        pltpu.async_copy(g_hbm.at[srcv.at[1]], rows1, sem1)

        def step2(i, _):
            b = i * 2
            pltpu.make_async_copy(g_hbm.at[srcv.at[b]], rows0, sem0).wait()
            pltpu.async_copy(rows0, acc.at[dstv.at[b]], sems0, add=True)
            pltpu.make_async_copy(g_hbm.at[srcv.at[b + 1]], rows1, sem1).wait()
            pltpu.async_copy(rows1, acc.at[dstv.at[b + 1]], sems1, add=True)

            pltpu.make_async_copy(rows0, acc.at[dstv.at[b]], sems0).wait()

            @pl.when(b + 2 < m)
            def _():
                pltpu.async_copy(g_hbm.at[srcv.at[b + 2]], rows0, sem0)

            pltpu.make_async_copy(rows1, acc.at[dstv.at[b + 1]], sems1).wait()

            @pl.when(b + 3 < m)
            def _():
                pltpu.async_copy(g_hbm.at[srcv.at[b + 3]], rows1, sem1)

            return 0

        lax.fori_loop(0, m // 2, step2, 0)

    def _run(base_blk, chunks):
        off = 0
        for m in chunks:
            _chunk(base_blk + off, m)
            off += m

    @pl.when(c == 0)
    def _():
        _run(s * B0, CH0)

    @pl.when(c == 1)
    def _():
        _run(NS * B0 + s * B1, CH1)

    plsc.subcore_barrier()

    for j in range(SLAB // K):
        r0 = s * SLAB + j * K
        pltpu.sync_copy(acc.at[pl.ds(r0, K)], rows0)
        pltpu.sync_copy(rows0, out.at[c, pl.ds(r0, K)])


_R = 2048
_G = NPAD // _R


def _b0_body(deg_ref, x_ref, w_ref, g_ref, dinv_ref):
    d = deg_ref[:, 0:1] + deg_ref[:, 1:2] + 1.0
    dinv = lax.rsqrt(d)
    m = jnp.dot(x_ref[...], w_ref[...], preferred_element_type=jnp.float32)
    g_ref[...] = dinv * m
    dinv_ref[...] = dinv


_b0_call = pl.pallas_call(
    _b0_body,
    grid=(_G,),
    in_specs=[
        pl.BlockSpec((_R, 2), lambda i: (i, 0)),
        pl.BlockSpec((_R, D), lambda i: (i, 0)),
        pl.BlockSpec((D, D), lambda i: (0, 0)),
    ],
    out_specs=[
        pl.BlockSpec((_R, D), lambda i: (i, 0)),
        pl.BlockSpec((_R, 1), lambda i: (i, 0)),
    ],
    out_shape=[
        jax.ShapeDtypeStruct((NPAD, D), jnp.float32),
        jax.ShapeDtypeStruct((NPAD, 1), jnp.float32),
    ],
)


def _mid_body(acc_ref, g_ref, dinv_ref, b_ref, w_ref, o_ref):
    accsum = acc_ref[0] + acc_ref[1]
    dinv = dinv_ref[...]
    h = jnp.maximum(dinv * (accsum + g_ref[...]) + b_ref[...], 0.0)
    o_ref[...] = dinv * jnp.dot(h, w_ref[...], preferred_element_type=jnp.float32)


_mid_call = pl.pallas_call(
    _mid_body,
    grid=(_G,),
    in_specs=[
        pl.BlockSpec((2, _R, D), lambda i: (0, i, 0)),
        pl.BlockSpec((_R, D), lambda i: (i, 0)),
        pl.BlockSpec((_R, 1), lambda i: (i, 0)),
        pl.BlockSpec((1, D), lambda i: (0, 0)),
        pl.BlockSpec((D, D), lambda i: (0, 0)),
    ],
    out_specs=pl.BlockSpec((_R, D), lambda i: (i, 0)),
    out_shape=jax.ShapeDtypeStruct((NPAD, D), jnp.float32),
)


def _final_body(acc_ref, g_ref, dinv_ref, b_ref, o_ref):
    accsum = acc_ref[0] + acc_ref[1]
    o_ref[...] = dinv_ref[...] * (accsum + g_ref[...]) + b_ref[...]


_final_call = pl.pallas_call(
    _final_body,
    grid=(_G,),
    in_specs=[
        pl.BlockSpec((2, _R, D), lambda i: (0, i, 0)),
        pl.BlockSpec((_R, D), lambda i: (i, 0)),
        pl.BlockSpec((_R, 1), lambda i: (i, 0)),
        pl.BlockSpec((1, D), lambda i: (0, 0)),
    ],
    out_specs=pl.BlockSpec((_R, D), lambda i: (i, 0)),
    out_shape=jax.ShapeDtypeStruct((NPAD, D), jnp.float32),
)


def kernel(x, edge_index, W0, b0, W1, b1, W2, b2):
    src = edge_index[0].astype(jnp.int32)
    dst = edge_index[1].astype(jnp.int32)
    epad = NBLOCKS_AL * K - E
    src3 = jnp.concatenate([src, jnp.zeros((epad,), jnp.int32)]).reshape(
        NBLOCKS_AL, K)
    dst3 = jnp.concatenate([dst, jnp.full((epad,), N, jnp.int32)]).reshape(
        NBLOCKS_AL, K)
    xp = jnp.pad(x, ((0, NPAD - N), (0, 0)))
    zeros_blk = jnp.zeros((NW, K, D), jnp.float32)
    b0_2 = b0[None, :]
    b1_2 = b1[None, :]
    b2_2 = b2[None, :]

    degp = _deg_kernel(dst3)
    deg_t = degp.T
    g0, dinv = _b0_call(deg_t, xp, W0)
    acc0 = _agg_kernel(g0, src3, dst3, zeros_blk)
    g1 = _mid_call(acc0, g0, dinv, b0_2, W1)
    acc1 = _agg_kernel(g1, src3, dst3, zeros_blk)
    g2 = _mid_call(acc1, g1, dinv, b1_2, W2)
    acc2 = _agg_kernel(g2, src3, dst3, zeros_blk)
    out = _final_call(acc2, g2, dinv, b2_2)
    return out[:N]
